# Initial kernel scaffold; baseline (speedup 1.0000x reference)
import jax
import jax.numpy as jnp
from jax.experimental import pallas as pl


def kernel(x, w_router, w1, w2):
    raise NotImplementedError("write your pallas kernel here")


# speedup vs baseline: 1.1854x; 1.1854x over previous
import functools

import jax
import jax.numpy as jnp
from jax import lax
from jax.experimental import pallas as pl
from jax.experimental.pallas import tpu as pltpu
from jax.experimental.pallas import tpu_sc as plsc

E = 8
TOPK = 2
D = 1024
DFF = 2048
T = 2048
C = (2 * T * TOPK) // E
TRASH = E * C
BUF_ROWS = E * C + 8

NC, NS = 2, 16
NW = NC * NS
TPW = T // NW

BC = 256
LB = 512


def _router_body(x_ref, wr_ref, d0d_ref, d1d_ref, d0c_ref, d1c_ref,
                 w0_ref, w1_ref, cnt_ref):
    x = x_ref[...]
    wr = wr_ref[...]
    logits = lax.dot_general(x.astype(jnp.bfloat16), wr.astype(jnp.bfloat16),
                             (((1,), (0,)), ((), ())),
                             preferred_element_type=jnp.float32)

    iota_e = lax.broadcasted_iota(jnp.int32, (T, E), 1)
    neg_inf = jnp.float32(-jnp.inf)

    m1 = jnp.max(logits, axis=1, keepdims=True)
    e0 = jnp.min(jnp.where(logits == m1, iota_e, E), axis=1, keepdims=True)
    masked = jnp.where(iota_e == e0, neg_inf, logits)
    m2 = jnp.max(masked, axis=1, keepdims=True)
    e1 = jnp.min(jnp.where(masked == m2, iota_e, E), axis=1, keepdims=True)

    w0 = 1.0 / (1.0 + jnp.exp(m2 - m1))
    w1 = 1.0 - w0

    onehot0 = (iota_e == e0).astype(jnp.float32)
    onehot1 = (iota_e == e1).astype(jnp.float32)
    c01 = onehot0 + onehot1

    r = lax.broadcasted_iota(jnp.int32, (LB, LB), 0)
    cc = lax.broadcasted_iota(jnp.int32, (LB, LB), 1)
    ltri = (cc < r).astype(jnp.float32)
    tot = jnp.zeros((1, E), jnp.float32)
    parts = []
    for b in range(T // LB):
        blk = c01[b * LB:(b + 1) * LB]
        local = lax.dot_general(ltri, blk, (((1,), (0,)), ((), ())),
                                preferred_element_type=jnp.float32)
        parts.append(local + tot)
        tot = tot + jnp.sum(blk, axis=0, keepdims=True)
    excl = jnp.concatenate(parts, axis=0)

    pos0 = jnp.sum(excl * onehot0, axis=1, keepdims=True)
    pos1 = jnp.sum(excl * onehot1, axis=1, keepdims=True)
    p0 = pos0.astype(jnp.int32)
    p1 = pos1.astype(jnp.int32)
    valid0 = p0 < C
    valid1 = p1 < C

    base0 = e0 * C
    base1 = e1 * C
    d0d_ref[...] = jnp.where(valid0, base0 + p0, TRASH)
    d1d_ref[...] = jnp.where(valid1, base1 + p1, TRASH)
    d0c_ref[...] = jnp.where(valid0, base0 + p0, base0)
    d1c_ref[...] = jnp.where(valid1, base1 + p1, base1)
    w0_ref[...] = jnp.where(valid0, w0, 0.0)
    w1_ref[...] = jnp.where(valid1, w1, 0.0)
    cnt_ref[...] = tot.astype(jnp.int32)


def _router(x, w_router):
    outs = pl.pallas_call(
        _router_body,
        out_shape=(
            jax.ShapeDtypeStruct((T, 1), jnp.int32),
            jax.ShapeDtypeStruct((T, 1), jnp.int32),
            jax.ShapeDtypeStruct((T, 1), jnp.int32),
            jax.ShapeDtypeStruct((T, 1), jnp.int32),
            jax.ShapeDtypeStruct((T, 1), jnp.float32),
            jax.ShapeDtypeStruct((T, 1), jnp.float32),
            jax.ShapeDtypeStruct((1, E), jnp.int32),
        ),
    )(x, w_router)
    return outs


def _dispatch_body(x_hbm, d0_hbm, d1_hbm, buf_hbm, idx0_v, idx1_v, rows_v, sem):
    wid = lax.axis_index("s") * NC + lax.axis_index("c")
    base = wid * TPW
    pltpu.sync_copy(x_hbm.at[pl.ds(base, TPW)], rows_v)
    pltpu.sync_copy(d0_hbm.at[pl.ds(base, TPW)], idx0_v)
    pltpu.sync_copy(d1_hbm.at[pl.ds(base, TPW)], idx1_v)
    pltpu.async_copy(rows_v, buf_hbm.at[idx0_v], sem).wait()
    pltpu.async_copy(rows_v, buf_hbm.at[idx1_v], sem).wait()


def _dispatch(x, d0, d1):
    mesh = plsc.VectorSubcoreMesh(core_axis_name="c", subcore_axis_name="s")
    return pl.kernel(
        _dispatch_body,
        out_type=jax.ShapeDtypeStruct((BUF_ROWS, D), jnp.float32),
        mesh=mesh,
        scratch_types=[
            pltpu.VMEM((TPW,), jnp.int32),
            pltpu.VMEM((TPW,), jnp.int32),
            pltpu.VMEM((TPW, D), jnp.float32),
            pltpu.SemaphoreType.DMA,
        ],
    )(x, d0, d1)


def _mlp_body(cnt_ref, x_ref, w1_ref, w2_ref, o_ref):
    e = pl.program_id(0)
    cb = pl.program_id(1)

    @pl.when(cb * BC < cnt_ref[e])
    def _():
        xb = x_ref[0].astype(jnp.bfloat16)
        h = lax.dot_general(xb, w1_ref[0], (((1,), (1,)), ((), ())),
                            preferred_element_type=jnp.float32)
        gate = h[:, :DFF]
        up = h[:, DFF:]
        act = (gate * jax.nn.sigmoid(gate) * up).astype(jnp.bfloat16)
        o_ref[0] = lax.dot_general(act, w2_ref[0], (((1,), (1,)), ((), ())),
                                   preferred_element_type=jnp.float32)


def _mlp(counts, buf3, w1_bf, w2_bf):
    grid_spec = pltpu.PrefetchScalarGridSpec(
        num_scalar_prefetch=1,
        grid=(E, C // BC),
        in_specs=[
            pl.BlockSpec((1, BC, D), lambda e, cb, cnt: (e, cb, 0)),
            pl.BlockSpec((1, 2 * DFF, D), lambda e, cb, cnt: (e, 0, 0)),
            pl.BlockSpec((1, D, DFF), lambda e, cb, cnt: (e, 0, 0)),
        ],
        out_specs=pl.BlockSpec((1, BC, D), lambda e, cb, cnt: (e, cb, 0)),
    )
    return pl.pallas_call(
        _mlp_body,
        grid_spec=grid_spec,
        out_shape=jax.ShapeDtypeStruct((E, C, D), jnp.float32),
    )(counts, buf3, w1_bf, w2_bf)


def _gather_body(oute_hbm, d0_hbm, d1_hbm, y0_hbm, y1_hbm, idx_v, rows_v, sem):
    wid = lax.axis_index("s") * NC + lax.axis_index("c")
    base = wid * TPW
    pltpu.sync_copy(d0_hbm.at[pl.ds(base, TPW)], idx_v)
    pltpu.async_copy(oute_hbm.at[idx_v], rows_v, sem).wait()
    pltpu.sync_copy(rows_v, y0_hbm.at[pl.ds(base, TPW)])
    pltpu.sync_copy(d1_hbm.at[pl.ds(base, TPW)], idx_v)
    pltpu.async_copy(oute_hbm.at[idx_v], rows_v, sem).wait()
    pltpu.sync_copy(rows_v, y1_hbm.at[pl.ds(base, TPW)])


def _gather(oute, d0c, d1c):
    mesh = plsc.VectorSubcoreMesh(core_axis_name="c", subcore_axis_name="s")
    return pl.kernel(
        _gather_body,
        out_type=(
            jax.ShapeDtypeStruct((T, D), jnp.float32),
            jax.ShapeDtypeStruct((T, D), jnp.float32),
        ),
        mesh=mesh,
        scratch_types=[
            pltpu.VMEM((TPW,), jnp.int32),
            pltpu.VMEM((TPW, D), jnp.float32),
            pltpu.SemaphoreType.DMA,
        ],
    )(oute, d0c, d1c)


BT = 512


def _combine_body(y0_ref, y1_ref, w0_ref, w1_ref, o_ref):
    o_ref[...] = y0_ref[...] * w0_ref[...] + y1_ref[...] * w1_ref[...]


def _combine(y0, y1, w0, w1):
    return pl.pallas_call(
        _combine_body,
        grid=(T // BT,),
        in_specs=[
            pl.BlockSpec((BT, D), lambda i: (i, 0)),
            pl.BlockSpec((BT, D), lambda i: (i, 0)),
            pl.BlockSpec((BT, 1), lambda i: (i, 0)),
            pl.BlockSpec((BT, 1), lambda i: (i, 0)),
        ],
        out_specs=pl.BlockSpec((BT, D), lambda i: (i, 0)),
        out_shape=jax.ShapeDtypeStruct((T, D), jnp.float32),
    )(y0, y1, w0, w1)


def kernel(x, w_router, w1, w2):
    d0d, d1d, d0c, d1c, w0, w1w, cnts = _router(x, w_router)
    buf = _dispatch(x, d0d.reshape(T), d1d.reshape(T))
    oute = _mlp(cnts.reshape(E), buf[:E * C].reshape(E, C, D),
                w1.astype(jnp.bfloat16), w2.astype(jnp.bfloat16))
    y0, y1 = _gather(oute.reshape(E * C, D), d0c.reshape(T), d1c.reshape(T))
    return _combine(y0, y1, w0, w1w)


# speedup vs baseline: 1.2222x; 1.0311x over previous
import functools

import jax
import jax.numpy as jnp
from jax import lax
from jax.experimental import pallas as pl
from jax.experimental.pallas import tpu as pltpu
from jax.experimental.pallas import tpu_sc as plsc

E = 8
TOPK = 2
D = 1024
DFF = 2048
T = 2048
C = (2 * T * TOPK) // E
TRASH = E * C
BUF_ROWS = E * C + 8

NC, NS = 2, 16
NW = NC * NS
TPW = T // NW

BC = 256
LB = 512


def _router_body(x_ref, wr_ref, d0d_ref, d1d_ref, d0c_ref, d1c_ref,
                 w0_ref, w1_ref, cnt_ref):
    x = x_ref[...]
    wr = wr_ref[...]
    logits = lax.dot_general(x.astype(jnp.bfloat16), wr.astype(jnp.bfloat16),
                             (((1,), (0,)), ((), ())),
                             preferred_element_type=jnp.float32)

    iota_e = lax.broadcasted_iota(jnp.int32, (T, E), 1)
    neg_inf = jnp.float32(-jnp.inf)

    m1 = jnp.max(logits, axis=1, keepdims=True)
    e0 = jnp.min(jnp.where(logits == m1, iota_e, E), axis=1, keepdims=True)
    masked = jnp.where(iota_e == e0, neg_inf, logits)
    m2 = jnp.max(masked, axis=1, keepdims=True)
    e1 = jnp.min(jnp.where(masked == m2, iota_e, E), axis=1, keepdims=True)

    w0 = 1.0 / (1.0 + jnp.exp(m2 - m1))
    w1 = 1.0 - w0

    onehot0 = (iota_e == e0).astype(jnp.float32)
    onehot1 = (iota_e == e1).astype(jnp.float32)
    c01 = onehot0 + onehot1

    r = lax.broadcasted_iota(jnp.int32, (LB, LB), 0)
    cc = lax.broadcasted_iota(jnp.int32, (LB, LB), 1)
    ltri = (cc < r).astype(jnp.float32)
    tot = jnp.zeros((1, E), jnp.float32)
    parts = []
    for b in range(T // LB):
        blk = c01[b * LB:(b + 1) * LB]
        local = lax.dot_general(ltri, blk, (((1,), (0,)), ((), ())),
                                preferred_element_type=jnp.float32)
        parts.append(local + tot)
        tot = tot + jnp.sum(blk, axis=0, keepdims=True)
    excl = jnp.concatenate(parts, axis=0)

    pos0 = jnp.sum(excl * onehot0, axis=1, keepdims=True)
    pos1 = jnp.sum(excl * onehot1, axis=1, keepdims=True)
    p0 = pos0.astype(jnp.int32)
    p1 = pos1.astype(jnp.int32)
    valid0 = p0 < C
    valid1 = p1 < C

    base0 = e0 * C
    base1 = e1 * C
    d0d_ref[...] = jnp.where(valid0, base0 + p0, TRASH)
    d1d_ref[...] = jnp.where(valid1, base1 + p1, TRASH)
    d0c_ref[...] = jnp.where(valid0, base0 + p0, base0)
    d1c_ref[...] = jnp.where(valid1, base1 + p1, base1)
    w0_ref[...] = jnp.where(valid0, w0, 0.0)
    w1_ref[...] = jnp.where(valid1, w1, 0.0)
    cnt_ref[...] = tot.astype(jnp.int32)


def _router(x, w_router):
    outs = pl.pallas_call(
        _router_body,
        out_shape=(
            jax.ShapeDtypeStruct((T, 1), jnp.int32),
            jax.ShapeDtypeStruct((T, 1), jnp.int32),
            jax.ShapeDtypeStruct((T, 1), jnp.int32),
            jax.ShapeDtypeStruct((T, 1), jnp.int32),
            jax.ShapeDtypeStruct((T, 1), jnp.float32),
            jax.ShapeDtypeStruct((T, 1), jnp.float32),
            jax.ShapeDtypeStruct((1, E), jnp.int32),
        ),
    )(x, w_router)
    return outs


def _dispatch_body(x_hbm, d0_hbm, d1_hbm, buf_hbm, idx0_v, idx1_v, rows_v, sem):
    wid = lax.axis_index("s") * NC + lax.axis_index("c")
    base = wid * TPW
    pltpu.sync_copy(x_hbm.at[pl.ds(base, TPW)], rows_v)
    pltpu.sync_copy(d0_hbm.at[pl.ds(base, TPW)], idx0_v)
    pltpu.sync_copy(d1_hbm.at[pl.ds(base, TPW)], idx1_v)
    pltpu.async_copy(rows_v, buf_hbm.at[idx0_v], sem).wait()
    pltpu.async_copy(rows_v, buf_hbm.at[idx1_v], sem).wait()


def _dispatch(x, d0, d1):
    mesh = plsc.VectorSubcoreMesh(core_axis_name="c", subcore_axis_name="s")
    return pl.kernel(
        _dispatch_body,
        out_type=jax.ShapeDtypeStruct((BUF_ROWS, D), jnp.float32),
        mesh=mesh,
        scratch_types=[
            pltpu.VMEM((TPW,), jnp.int32),
            pltpu.VMEM((TPW,), jnp.int32),
            pltpu.VMEM((TPW, D), jnp.float32),
            pltpu.SemaphoreType.DMA,
        ],
    )(x, d0, d1)


FB = 512


def _mlp_body(cnt_ref, x_ref, wg_ref, wu_ref, w2_ref, o_ref):
    e = pl.program_id(0)
    f = pl.program_id(1)
    cnt = cnt_ref[e]
    for cb in range(C // BC):
        @pl.when(cb * BC < cnt)
        def _():
            sl = slice(cb * BC, (cb + 1) * BC)
            xb = x_ref[0, sl].astype(jnp.bfloat16)
            wg = wg_ref[0].astype(jnp.bfloat16)
            wu = wu_ref[0].astype(jnp.bfloat16)
            g = lax.dot_general(xb, wg, (((1,), (1,)), ((), ())),
                                preferred_element_type=jnp.float32)
            u = lax.dot_general(xb, wu, (((1,), (1,)), ((), ())),
                                preferred_element_type=jnp.float32)
            act = (g * jax.nn.sigmoid(g) * u).astype(jnp.bfloat16)
            w2b = w2_ref[0].astype(jnp.bfloat16)
            val = lax.dot_general(act, w2b, (((1,), (1,)), ((), ())),
                                  preferred_element_type=jnp.float32)

            @pl.when(f == 0)
            def _():
                o_ref[0, sl] = val

            @pl.when(f > 0)
            def _():
                o_ref[0, sl] += val


def _mlp(counts, buf3, w1, w2):
    w1g = w1[:, :DFF, :]
    w1u = w1[:, DFF:, :]
    grid_spec = pltpu.PrefetchScalarGridSpec(
        num_scalar_prefetch=1,
        grid=(E, DFF // FB),
        in_specs=[
            pl.BlockSpec((1, C, D), lambda e, f, cnt: (e, 0, 0)),
            pl.BlockSpec((1, FB, D), lambda e, f, cnt: (e, f, 0)),
            pl.BlockSpec((1, FB, D), lambda e, f, cnt: (e, f, 0)),
            pl.BlockSpec((1, D, FB), lambda e, f, cnt: (e, 0, f)),
        ],
        out_specs=pl.BlockSpec((1, C, D), lambda e, f, cnt: (e, 0, 0)),
    )
    return pl.pallas_call(
        _mlp_body,
        grid_spec=grid_spec,
        out_shape=jax.ShapeDtypeStruct((E, C, D), jnp.float32),
    )(counts, buf3, w1g, w1u, w2)


def _gather_body(oute_hbm, d0_hbm, d1_hbm, y0_hbm, y1_hbm, idx_v, rows_v, sem):
    wid = lax.axis_index("s") * NC + lax.axis_index("c")
    base = wid * TPW
    pltpu.sync_copy(d0_hbm.at[pl.ds(base, TPW)], idx_v)
    pltpu.async_copy(oute_hbm.at[idx_v], rows_v, sem).wait()
    pltpu.sync_copy(rows_v, y0_hbm.at[pl.ds(base, TPW)])
    pltpu.sync_copy(d1_hbm.at[pl.ds(base, TPW)], idx_v)
    pltpu.async_copy(oute_hbm.at[idx_v], rows_v, sem).wait()
    pltpu.sync_copy(rows_v, y1_hbm.at[pl.ds(base, TPW)])


def _gather(oute, d0c, d1c):
    mesh = plsc.VectorSubcoreMesh(core_axis_name="c", subcore_axis_name="s")
    return pl.kernel(
        _gather_body,
        out_type=(
            jax.ShapeDtypeStruct((T, D), jnp.float32),
            jax.ShapeDtypeStruct((T, D), jnp.float32),
        ),
        mesh=mesh,
        scratch_types=[
            pltpu.VMEM((TPW,), jnp.int32),
            pltpu.VMEM((TPW, D), jnp.float32),
            pltpu.SemaphoreType.DMA,
        ],
    )(oute, d0c, d1c)


BT = 512


def _combine_body(y0_ref, y1_ref, w0_ref, w1_ref, o_ref):
    o_ref[...] = y0_ref[...] * w0_ref[...] + y1_ref[...] * w1_ref[...]


def _combine(y0, y1, w0, w1):
    return pl.pallas_call(
        _combine_body,
        grid=(T // BT,),
        in_specs=[
            pl.BlockSpec((BT, D), lambda i: (i, 0)),
            pl.BlockSpec((BT, D), lambda i: (i, 0)),
            pl.BlockSpec((BT, 1), lambda i: (i, 0)),
            pl.BlockSpec((BT, 1), lambda i: (i, 0)),
        ],
        out_specs=pl.BlockSpec((BT, D), lambda i: (i, 0)),
        out_shape=jax.ShapeDtypeStruct((T, D), jnp.float32),
    )(y0, y1, w0, w1)


def kernel(x, w_router, w1, w2):
    d0d, d1d, d0c, d1c, w0, w1w, cnts = _router(x, w_router)
    buf = _dispatch(x, d0d.reshape(T), d1d.reshape(T))
    oute = _mlp(cnts.reshape(E), buf[:E * C].reshape(E, C, D), w1, w2)
    y0, y1 = _gather(oute.reshape(E * C, D), d0c.reshape(T), d1c.reshape(T))
    return _combine(y0, y1, w0, w1w)


# speedup vs baseline: 1.8740x; 1.5333x over previous
import functools

import jax
import jax.numpy as jnp
from jax import lax
from jax.experimental import pallas as pl
from jax.experimental.pallas import tpu as pltpu
from jax.experimental.pallas import tpu_sc as plsc

E = 8
TOPK = 2
D = 1024
DFF = 2048
T = 2048
C = (2 * T * TOPK) // E
TRASH = E * C
BUF_ROWS = (E + 1) * C

NC, NS = 2, 16
NW = NC * NS
TPW = T // NW

BC = 256
LB = 512


def _router_body(x_ref, wr_ref, d0d_ref, d1d_ref, d0c_ref, d1c_ref,
                 w0_ref, w1_ref, cnt_ref):
    x = x_ref[...]
    wr = wr_ref[...]
    logits = lax.dot_general(x.astype(jnp.bfloat16), wr.astype(jnp.bfloat16),
                             (((1,), (0,)), ((), ())),
                             preferred_element_type=jnp.float32)

    iota_e = lax.broadcasted_iota(jnp.int32, (T, E), 1)
    neg_inf = jnp.float32(-jnp.inf)

    m1 = jnp.max(logits, axis=1, keepdims=True)
    e0 = jnp.min(jnp.where(logits == m1, iota_e, E), axis=1, keepdims=True)
    masked = jnp.where(iota_e == e0, neg_inf, logits)
    m2 = jnp.max(masked, axis=1, keepdims=True)
    e1 = jnp.min(jnp.where(masked == m2, iota_e, E), axis=1, keepdims=True)

    w0 = 1.0 / (1.0 + jnp.exp(m2 - m1))
    w1 = 1.0 - w0

    onehot0 = (iota_e == e0).astype(jnp.float32)
    onehot1 = (iota_e == e1).astype(jnp.float32)
    c01 = onehot0 + onehot1

    r = lax.broadcasted_iota(jnp.int32, (LB, LB), 0)
    cc = lax.broadcasted_iota(jnp.int32, (LB, LB), 1)
    ltri = (cc < r).astype(jnp.float32)
    tot = jnp.zeros((1, E), jnp.float32)
    parts = []
    for b in range(T // LB):
        blk = c01[b * LB:(b + 1) * LB]
        local = lax.dot_general(ltri, blk, (((1,), (0,)), ((), ())),
                                preferred_element_type=jnp.float32)
        parts.append(local + tot)
        tot = tot + jnp.sum(blk, axis=0, keepdims=True)
    excl = jnp.concatenate(parts, axis=0)

    pos0 = jnp.sum(excl * onehot0, axis=1, keepdims=True)
    pos1 = jnp.sum(excl * onehot1, axis=1, keepdims=True)
    p0 = pos0.astype(jnp.int32)
    p1 = pos1.astype(jnp.int32)
    valid0 = p0 < C
    valid1 = p1 < C

    base0 = e0 * C
    base1 = e1 * C
    d0d_ref[...] = jnp.where(valid0, base0 + p0, TRASH)
    d1d_ref[...] = jnp.where(valid1, base1 + p1, TRASH)
    d0c_ref[...] = jnp.where(valid0, base0 + p0, base0)
    d1c_ref[...] = jnp.where(valid1, base1 + p1, base1)
    w0_ref[...] = jnp.where(valid0, w0, 0.0)
    w1_ref[...] = jnp.where(valid1, w1, 0.0)
    cnt_ref[...] = tot.astype(jnp.int32)


def _router(x, w_router):
    outs = pl.pallas_call(
        _router_body,
        out_shape=(
            jax.ShapeDtypeStruct((T, 1), jnp.int32),
            jax.ShapeDtypeStruct((T, 1), jnp.int32),
            jax.ShapeDtypeStruct((T, 1), jnp.int32),
            jax.ShapeDtypeStruct((T, 1), jnp.int32),
            jax.ShapeDtypeStruct((T, 1), jnp.float32),
            jax.ShapeDtypeStruct((T, 1), jnp.float32),
            jax.ShapeDtypeStruct((1, E), jnp.int32),
        ),
    )(x, w_router)
    return outs


def _dispatch_body(x_hbm, d0_hbm, d1_hbm, buf_hbm, idx0_v, idx1_v, rows_v, sem):
    wid = lax.axis_index("s") * NC + lax.axis_index("c")
    base = wid * TPW
    pltpu.sync_copy(x_hbm.at[pl.ds(base, TPW)], rows_v)
    pltpu.sync_copy(d0_hbm.at[pl.ds(base, TPW)], idx0_v)
    pltpu.sync_copy(d1_hbm.at[pl.ds(base, TPW)], idx1_v)
    pltpu.async_copy(rows_v, buf_hbm.at[idx0_v], sem).wait()
    pltpu.async_copy(rows_v, buf_hbm.at[idx1_v], sem).wait()


def _dispatch(x, d0, d1):
    mesh = plsc.VectorSubcoreMesh(core_axis_name="c", subcore_axis_name="s")
    return pl.kernel(
        _dispatch_body,
        out_type=jax.ShapeDtypeStruct((BUF_ROWS, D), jnp.float32),
        mesh=mesh,
        scratch_types=[
            pltpu.VMEM((TPW,), jnp.int32),
            pltpu.VMEM((TPW,), jnp.int32),
            pltpu.VMEM((TPW, D), jnp.float32),
            pltpu.SemaphoreType.DMA,
        ],
    )(x, d0, d1)


FB = 512


def _mlp_body(cnt_ref, x_ref, wg_ref, wu_ref, w2_ref, o_ref):
    e = pl.program_id(0)
    f = pl.program_id(1)
    cnt = cnt_ref[e]
    for cb in range(C // BC):
        @pl.when(cb * BC < cnt)
        def _():
            sl = slice(cb * BC, (cb + 1) * BC)
            xb = x_ref[0, sl].astype(jnp.bfloat16)
            wg = wg_ref[0].astype(jnp.bfloat16)
            wu = wu_ref[0].astype(jnp.bfloat16)
            g = lax.dot_general(xb, wg, (((1,), (1,)), ((), ())),
                                preferred_element_type=jnp.float32)
            u = lax.dot_general(xb, wu, (((1,), (1,)), ((), ())),
                                preferred_element_type=jnp.float32)
            act = (g * jax.nn.sigmoid(g) * u).astype(jnp.bfloat16)
            w2b = w2_ref[0].astype(jnp.bfloat16)
            val = lax.dot_general(act, w2b, (((1,), (1,)), ((), ())),
                                  preferred_element_type=jnp.float32)

            @pl.when(f == 0)
            def _():
                o_ref[0, sl] = val

            @pl.when(f > 0)
            def _():
                o_ref[0, sl] += val


def _mlp(counts, buf3, w1, w2):
    nf = DFF // FB
    grid_spec = pltpu.PrefetchScalarGridSpec(
        num_scalar_prefetch=1,
        grid=(E, nf),
        in_specs=[
            pl.BlockSpec((1, C, D), lambda e, f, cnt: (e, 0, 0)),
            pl.BlockSpec((1, FB, D), lambda e, f, cnt: (e, f, 0)),
            pl.BlockSpec((1, FB, D), lambda e, f, cnt: (e, f + nf, 0)),
            pl.BlockSpec((1, D, FB), lambda e, f, cnt: (e, 0, f)),
        ],
        out_specs=pl.BlockSpec((1, C, D), lambda e, f, cnt: (e, 0, 0)),
    )
    return pl.pallas_call(
        _mlp_body,
        grid_spec=grid_spec,
        out_shape=jax.ShapeDtypeStruct((E, C, D), jnp.float32),
    )(counts, buf3, w1, w1, w2)


def _gather_body(oute_hbm, d0_hbm, d1_hbm, y0_hbm, y1_hbm, idx_v, rows_v, sem):
    wid = lax.axis_index("s") * NC + lax.axis_index("c")
    base = wid * TPW
    pltpu.sync_copy(d0_hbm.at[pl.ds(base, TPW)], idx_v)
    pltpu.async_copy(oute_hbm.at[idx_v], rows_v, sem).wait()
    pltpu.sync_copy(rows_v, y0_hbm.at[pl.ds(base, TPW)])
    pltpu.sync_copy(d1_hbm.at[pl.ds(base, TPW)], idx_v)
    pltpu.async_copy(oute_hbm.at[idx_v], rows_v, sem).wait()
    pltpu.sync_copy(rows_v, y1_hbm.at[pl.ds(base, TPW)])


def _gather(oute, d0c, d1c):
    mesh = plsc.VectorSubcoreMesh(core_axis_name="c", subcore_axis_name="s")
    return pl.kernel(
        _gather_body,
        out_type=(
            jax.ShapeDtypeStruct((T, D), jnp.float32),
            jax.ShapeDtypeStruct((T, D), jnp.float32),
        ),
        mesh=mesh,
        scratch_types=[
            pltpu.VMEM((TPW,), jnp.int32),
            pltpu.VMEM((TPW, D), jnp.float32),
            pltpu.SemaphoreType.DMA,
        ],
    )(oute, d0c, d1c)


BT = 512


def _combine_body(y0_ref, y1_ref, w0_ref, w1_ref, o_ref):
    o_ref[...] = y0_ref[...] * w0_ref[...] + y1_ref[...] * w1_ref[...]


def _combine(y0, y1, w0, w1):
    return pl.pallas_call(
        _combine_body,
        grid=(T // BT,),
        in_specs=[
            pl.BlockSpec((BT, D), lambda i: (i, 0)),
            pl.BlockSpec((BT, D), lambda i: (i, 0)),
            pl.BlockSpec((BT, 1), lambda i: (i, 0)),
            pl.BlockSpec((BT, 1), lambda i: (i, 0)),
        ],
        out_specs=pl.BlockSpec((BT, D), lambda i: (i, 0)),
        out_shape=jax.ShapeDtypeStruct((T, D), jnp.float32),
    )(y0, y1, w0, w1)


def kernel(x, w_router, w1, w2):
    d0d, d1d, d0c, d1c, w0, w1w, cnts = _router(x, w_router)
    buf = _dispatch(x, d0d.reshape(T), d1d.reshape(T))
    oute = _mlp(cnts.reshape(E), buf.reshape(E + 1, C, D), w1, w2)
    y0, y1 = _gather(oute.reshape(E * C, D), d0c.reshape(T), d1c.reshape(T))
    return _combine(y0, y1, w0, w1w)


# speedup vs baseline: 2.1217x; 1.1322x over previous
import functools

import jax
import jax.numpy as jnp
from jax import lax
from jax.experimental import pallas as pl
from jax.experimental.pallas import tpu as pltpu
from jax.experimental.pallas import tpu_sc as plsc

E = 8
TOPK = 2
D = 1024
DFF = 2048
T = 2048
C = (2 * T * TOPK) // E
TRASH = E * C
BUF_ROWS = (E + 1) * C

NC, NS = 2, 16
NW = NC * NS
TPW = T // NW

BC = 256
LB = 512


def _router_body(x_ref, wr_ref, d0d_ref, d1d_ref, d0c_ref, d1c_ref,
                 w0_ref, w1_ref, cnt_ref):
    x = x_ref[...]
    wr = wr_ref[...]
    logits = lax.dot_general(x.astype(jnp.bfloat16), wr.astype(jnp.bfloat16),
                             (((1,), (0,)), ((), ())),
                             preferred_element_type=jnp.float32)

    iota_e = lax.broadcasted_iota(jnp.int32, (T, E), 1)
    neg_inf = jnp.float32(-jnp.inf)

    m1 = jnp.max(logits, axis=1, keepdims=True)
    e0 = jnp.min(jnp.where(logits == m1, iota_e, E), axis=1, keepdims=True)
    masked = jnp.where(iota_e == e0, neg_inf, logits)
    m2 = jnp.max(masked, axis=1, keepdims=True)
    e1 = jnp.min(jnp.where(masked == m2, iota_e, E), axis=1, keepdims=True)

    w0 = 1.0 / (1.0 + jnp.exp(m2 - m1))
    w1 = 1.0 - w0

    onehot0 = (iota_e == e0).astype(jnp.float32)
    onehot1 = (iota_e == e1).astype(jnp.float32)
    c01 = onehot0 + onehot1

    r = lax.broadcasted_iota(jnp.int32, (LB, LB), 0)
    cc = lax.broadcasted_iota(jnp.int32, (LB, LB), 1)
    ltri = (cc < r).astype(jnp.float32)
    tot = jnp.zeros((1, E), jnp.float32)
    parts = []
    for b in range(T // LB):
        blk = c01[b * LB:(b + 1) * LB]
        local = lax.dot_general(ltri, blk, (((1,), (0,)), ((), ())),
                                preferred_element_type=jnp.float32)
        parts.append(local + tot)
        tot = tot + jnp.sum(blk, axis=0, keepdims=True)
    excl = jnp.concatenate(parts, axis=0)

    pos0 = jnp.sum(excl * onehot0, axis=1, keepdims=True)
    pos1 = jnp.sum(excl * onehot1, axis=1, keepdims=True)
    p0 = pos0.astype(jnp.int32)
    p1 = pos1.astype(jnp.int32)
    valid0 = p0 < C
    valid1 = p1 < C

    base0 = e0 * C
    base1 = e1 * C
    d0d_ref[...] = jnp.where(valid0, base0 + p0, TRASH)
    d1d_ref[...] = jnp.where(valid1, base1 + p1, TRASH)
    d0c_ref[...] = jnp.where(valid0, base0 + p0, base0)
    d1c_ref[...] = jnp.where(valid1, base1 + p1, base1)
    w0_ref[...] = jnp.where(valid0, w0, 0.0)
    w1_ref[...] = jnp.where(valid1, w1, 0.0)
    cnt_ref[...] = tot.astype(jnp.int32)


def _router(x, w_router):
    outs = pl.pallas_call(
        _router_body,
        out_shape=(
            jax.ShapeDtypeStruct((T, 1), jnp.int32),
            jax.ShapeDtypeStruct((T, 1), jnp.int32),
            jax.ShapeDtypeStruct((T, 1), jnp.int32),
            jax.ShapeDtypeStruct((T, 1), jnp.int32),
            jax.ShapeDtypeStruct((T, 1), jnp.float32),
            jax.ShapeDtypeStruct((T, 1), jnp.float32),
            jax.ShapeDtypeStruct((1, E), jnp.int32),
        ),
    )(x, w_router)
    return outs


def _dispatch_body(x_hbm, d0_hbm, d1_hbm, buf_hbm, idx0_v, idx1_v, rows_v, sem):
    wid = lax.axis_index("s") * NC + lax.axis_index("c")
    base = wid * TPW
    pltpu.sync_copy(x_hbm.at[pl.ds(base, TPW)], rows_v)
    pltpu.sync_copy(d0_hbm.at[pl.ds(base, TPW)], idx0_v)
    pltpu.sync_copy(d1_hbm.at[pl.ds(base, TPW)], idx1_v)
    pltpu.async_copy(rows_v, buf_hbm.at[idx0_v], sem).wait()
    pltpu.async_copy(rows_v, buf_hbm.at[idx1_v], sem).wait()


def _dispatch(x, d0, d1):
    mesh = plsc.VectorSubcoreMesh(core_axis_name="c", subcore_axis_name="s")
    return pl.kernel(
        _dispatch_body,
        out_type=jax.ShapeDtypeStruct((BUF_ROWS, D), jnp.float32),
        mesh=mesh,
        scratch_types=[
            pltpu.VMEM((TPW,), jnp.int32),
            pltpu.VMEM((TPW,), jnp.int32),
            pltpu.VMEM((TPW, D), jnp.float32),
            pltpu.SemaphoreType.DMA,
        ],
    )(x, d0, d1)


FB = 1024


def _mlp_body(cnt_ref, x_ref, wg_ref, wu_ref, w2_ref, o_ref):
    e = pl.program_id(0)
    f = pl.program_id(1)
    cnt = cnt_ref[e]
    for cb in range(C // BC):
        @pl.when(cb * BC < cnt)
        def _():
            sl = slice(cb * BC, (cb + 1) * BC)
            xb = x_ref[0, sl].astype(jnp.bfloat16)
            wg = wg_ref[0].astype(jnp.bfloat16)
            wu = wu_ref[0].astype(jnp.bfloat16)
            g = lax.dot_general(xb, wg, (((1,), (1,)), ((), ())),
                                preferred_element_type=jnp.float32)
            u = lax.dot_general(xb, wu, (((1,), (1,)), ((), ())),
                                preferred_element_type=jnp.float32)
            act = (g * jax.nn.sigmoid(g) * u).astype(jnp.bfloat16)
            w2b = w2_ref[0].astype(jnp.bfloat16)
            val = lax.dot_general(act, w2b, (((1,), (1,)), ((), ())),
                                  preferred_element_type=jnp.float32)

            @pl.when(f == 0)
            def _():
                o_ref[0, sl] = val

            @pl.when(f > 0)
            def _():
                o_ref[0, sl] += val


def _mlp(counts, buf3, w1, w2):
    nf = DFF // FB
    grid_spec = pltpu.PrefetchScalarGridSpec(
        num_scalar_prefetch=1,
        grid=(E, nf),
        in_specs=[
            pl.BlockSpec((1, C, D), lambda e, f, cnt: (e, 0, 0)),
            pl.BlockSpec((1, FB, D), lambda e, f, cnt: (e, f, 0)),
            pl.BlockSpec((1, FB, D), lambda e, f, cnt: (e, f + nf, 0)),
            pl.BlockSpec((1, D, FB), lambda e, f, cnt: (e, 0, f)),
        ],
        out_specs=pl.BlockSpec((1, C, D), lambda e, f, cnt: (e, 0, 0)),
    )
    return pl.pallas_call(
        _mlp_body,
        grid_spec=grid_spec,
        out_shape=jax.ShapeDtypeStruct((E, C, D), jnp.float32),
    )(counts, buf3, w1, w1, w2)


def _gather_body(oute_hbm, d0_hbm, d1_hbm, y0_hbm, y1_hbm, idx_v, rows_v, sem):
    wid = lax.axis_index("s") * NC + lax.axis_index("c")
    base = wid * TPW
    pltpu.sync_copy(d0_hbm.at[pl.ds(base, TPW)], idx_v)
    pltpu.async_copy(oute_hbm.at[idx_v], rows_v, sem).wait()
    pltpu.sync_copy(rows_v, y0_hbm.at[pl.ds(base, TPW)])
    pltpu.sync_copy(d1_hbm.at[pl.ds(base, TPW)], idx_v)
    pltpu.async_copy(oute_hbm.at[idx_v], rows_v, sem).wait()
    pltpu.sync_copy(rows_v, y1_hbm.at[pl.ds(base, TPW)])


def _gather(oute, d0c, d1c):
    mesh = plsc.VectorSubcoreMesh(core_axis_name="c", subcore_axis_name="s")
    return pl.kernel(
        _gather_body,
        out_type=(
            jax.ShapeDtypeStruct((T, D), jnp.float32),
            jax.ShapeDtypeStruct((T, D), jnp.float32),
        ),
        mesh=mesh,
        scratch_types=[
            pltpu.VMEM((TPW,), jnp.int32),
            pltpu.VMEM((TPW, D), jnp.float32),
            pltpu.SemaphoreType.DMA,
        ],
    )(oute, d0c, d1c)


BT = 512


def _combine_body(y0_ref, y1_ref, w0_ref, w1_ref, o_ref):
    o_ref[...] = y0_ref[...] * w0_ref[...] + y1_ref[...] * w1_ref[...]


def _combine(y0, y1, w0, w1):
    return pl.pallas_call(
        _combine_body,
        grid=(T // BT,),
        in_specs=[
            pl.BlockSpec((BT, D), lambda i: (i, 0)),
            pl.BlockSpec((BT, D), lambda i: (i, 0)),
            pl.BlockSpec((BT, 1), lambda i: (i, 0)),
            pl.BlockSpec((BT, 1), lambda i: (i, 0)),
        ],
        out_specs=pl.BlockSpec((BT, D), lambda i: (i, 0)),
        out_shape=jax.ShapeDtypeStruct((T, D), jnp.float32),
    )(y0, y1, w0, w1)


def kernel(x, w_router, w1, w2):
    d0d, d1d, d0c, d1c, w0, w1w, cnts = _router(x, w_router)
    buf = _dispatch(x, d0d.reshape(T), d1d.reshape(T))
    oute = _mlp(cnts.reshape(E), buf.reshape(E + 1, C, D), w1, w2)
    y0, y1 = _gather(oute.reshape(E * C, D), d0c.reshape(T), d1c.reshape(T))
    return _combine(y0, y1, w0, w1w)
